# baseline (speedup 1.0000x reference)
import jax
import jax.numpy as jnp
from jax import lax
from jax.experimental import pallas as pl
from jax.experimental.pallas import tpu as pltpu
from jax.experimental.pallas import tpu_sc as plsc

N = 10000
D = 128
E = 320000
NUM_FEAT = 21
NUM_CLASSES = 6
NUM_GRAPHS = 64

NC = 2
NS = 16
NW = NC * NS
IC = 128
RI = 80
EP = NW * RI * IC
N_PAD = 10240
ROWS_PER_TILE = N_PAD // NS
NCH = 5
RCH = RI // NCH

NWR = NW * RI
R0 = 48
R1 = 112

BLK = 1000
NBLK = N // BLK

_mesh = plsc.VectorSubcoreMesh(core_axis_name="c", subcore_axis_name="s")


HF = IC // 2


def _sc_scatter_body(mp_hbm, src_hbm, dst_hbm, out_hbm,
                     sidx, didx, didx64, ga, gb, acc_sh, sema, semb):
    c = lax.axis_index("c")
    s = lax.axis_index("s")
    w = c * NS + s
    row0 = s * ROWS_PER_TILE

    z16 = jnp.zeros((16,), jnp.float32)

    def zero_body(i, carry):
        ga[i // 8, pl.ds((i % 8) * 16, 16)] = z16
        return carry

    lax.fori_loop(0, HF * (D // 16), zero_body, 0)
    for t in range(ROWS_PER_TILE // HF):
        pltpu.sync_copy(ga, acc_sh.at[pl.ds(row0 + t * HF, HF)])
    plsc.subcore_barrier()

    def edge_phase(base, nrows):
      for ch in range(nrows // RCH):
        pltpu.sync_copy(src_hbm.at[pl.ds(base + ch * RCH, RCH)], sidx)
        pltpu.sync_copy(dst_hbm.at[pl.ds(base + ch * RCH, RCH)], didx)

        def pack_body(j, carry):
            didx64[j // 4, pl.ds((j % 4) * 16, 16)] = (
                didx[j // 8, pl.ds((j % 8) * 16, 16)])
            return carry

        lax.fori_loop(0, RCH * (IC // 16), pack_body, 0)

        def src_at(i, h):
            return mp_hbm.at[sidx.at[i, pl.ds(h * HF, HF)]]

        pltpu.async_copy(src_at(0, 0), ga, sema)
        pltpu.async_copy(src_at(0, 1), gb, semb)

        def edge_body(i, carry):
            pltpu.make_async_copy(src_at(i, 0), ga, sema).wait()
            pltpu.sync_copy(ga, acc_sh.at[didx64.at[2 * i]], add=True)

            @pl.when(i < RCH - 1)
            def _():
                pltpu.async_copy(src_at(i + 1, 0), ga, sema)

            pltpu.make_async_copy(src_at(i, 1), gb, semb).wait()
            pltpu.sync_copy(gb, acc_sh.at[didx64.at[2 * i + 1]], add=True)

            @pl.when(i < RCH - 1)
            def _():
                pltpu.async_copy(src_at(i + 1, 1), gb, semb)

            return carry

        lax.fori_loop(0, RCH, edge_body, 0)

    @pl.when(c == 0)
    def _():
        edge_phase(s * R0, R0)

    @pl.when(c == 1)
    def _():
        edge_phase(NS * R0 + s * R1, R1)

    plsc.subcore_barrier()

    for t in range(ROWS_PER_TILE // HF):
        pltpu.sync_copy(acc_sh.at[pl.ds(row0 + t * HF, HF)], ga)
        pltpu.sync_copy(ga, out_hbm.at[c, pl.ds(row0 + t * HF, HF)])


_sc_scatter = pl.kernel(
    _sc_scatter_body,
    out_type=jax.ShapeDtypeStruct((NC, N_PAD, D), jnp.float32),
    mesh=_mesh,
    scratch_types=[
        pltpu.VMEM((RCH, IC), jnp.int32),
        pltpu.VMEM((RCH, IC), jnp.int32),
        pltpu.VMEM((2 * RCH, HF), jnp.int32),
        pltpu.VMEM((HF, D), jnp.float32),
        pltpu.VMEM((HF, D), jnp.float32),
        pltpu.VMEM_SHARED((N_PAD, D), jnp.float32),
        pltpu.SemaphoreType.DMA,
        pltpu.SemaphoreType.DMA,
    ],
)


def _sc_ones_body(dst_hbm, out_hbm, didx, gbuf, acc_sh):
    c = lax.axis_index("c")
    s = lax.axis_index("s")
    w = c * NS + s
    row0 = s * ROWS_PER_TILE

    z16 = jnp.zeros((16,), jnp.float32)

    def zero_body(i, carry):
        gbuf[i // 8, pl.ds((i % 8) * 16, 16)] = z16
        return carry

    lax.fori_loop(0, IC * (D // 16), zero_body, 0)
    for t in range(ROWS_PER_TILE // IC):
        pltpu.sync_copy(gbuf, acc_sh.at[pl.ds(row0 + t * IC, IC)])
    plsc.subcore_barrier()

    one16 = jnp.ones((16,), jnp.float32)

    def ones_body(i, carry):
        gbuf[i // 8, pl.ds((i % 8) * 16, 16)] = one16
        return carry

    lax.fori_loop(0, IC * (D // 16), ones_body, 0)
    for ch in range(NCH):
        pltpu.sync_copy(dst_hbm.at[pl.ds(w * RI + ch * RCH, RCH)], didx)

        def edge_body(r, carry):
            pltpu.sync_copy(gbuf, acc_sh.at[didx.at[r]], add=True)
            return carry

        lax.fori_loop(0, RCH, edge_body, 0)
    plsc.subcore_barrier()

    for t in range(ROWS_PER_TILE // IC):
        pltpu.sync_copy(acc_sh.at[pl.ds(row0 + t * IC, IC)], gbuf)
        pltpu.sync_copy(gbuf, out_hbm.at[c, pl.ds(row0 + t * IC, IC)])


_sc_ones = pl.kernel(
    _sc_ones_body,
    out_type=jax.ShapeDtypeStruct((NC, N_PAD, D), jnp.float32),
    mesh=_mesh,
    scratch_types=[
        pltpu.VMEM((RCH, IC), jnp.int32),
        pltpu.VMEM((IC, D), jnp.float32),
        pltpu.VMEM_SHARED((N_PAD, D), jnp.float32),
    ],
)


def _dinv_body(deg_ref, out_ref):
    cnt = jnp.sum(deg_ref[0] + deg_ref[1], axis=1) * (1.0 / D)
    out_ref[0, 0, :] = lax.rsqrt(cnt + 1.0)


_tc_dinv = pl.pallas_call(
    _dinv_body,
    grid=(NBLK,),
    in_specs=[pl.BlockSpec((NC, BLK, D), lambda i: (0, i, 0))],
    out_specs=pl.BlockSpec((1, 1, BLK), lambda i: (i, 0, 0)),
    out_shape=jax.ShapeDtypeStruct((NBLK, 1, BLK), jnp.float32),
)


def _pre_body(x_ref, dinv_ref, emb_ref, w1_ref, mp_ref):
    xb = x_ref[0, 0, :]
    oh = (xb[:, None] == lax.broadcasted_iota(jnp.int32, (BLK, NUM_FEAT), 1))
    h0 = jnp.dot(oh.astype(jnp.float32), emb_ref[...],
                 preferred_element_type=jnp.float32)
    dinv = dinv_ref[0, 0, :]
    mp_ref[...] = jnp.dot(h0, w1_ref[...],
                          preferred_element_type=jnp.float32) * dinv[:, None]


_tc_pre = pl.pallas_call(
    _pre_body,
    grid=(NBLK,),
    in_specs=[
        pl.BlockSpec((1, 1, BLK), lambda i: (i, 0, 0)),
        pl.BlockSpec((1, 1, BLK), lambda i: (i, 0, 0)),
        pl.BlockSpec((NUM_FEAT, D), lambda i: (0, 0)),
        pl.BlockSpec((D, D), lambda i: (0, 0)),
    ],
    out_specs=pl.BlockSpec((BLK, D), lambda i: (i, 0)),
    out_shape=jax.ShapeDtypeStruct((N, D), jnp.float32),
)


def _mid_body(s_ref, mp_ref, dinv_ref, b_ref, w_ref, out_ref):
    stot = s_ref[0] + s_ref[1] + mp_ref[...]
    dinv = dinv_ref[0, 0, :]
    h = jnp.maximum(dinv[:, None] * stot + b_ref[...], 0.0)
    out_ref[...] = jnp.dot(h, w_ref[...],
                           preferred_element_type=jnp.float32) * dinv[:, None]


_tc_mid = pl.pallas_call(
    _mid_body,
    grid=(NBLK,),
    in_specs=[
        pl.BlockSpec((NC, BLK, D), lambda i: (0, i, 0)),
        pl.BlockSpec((BLK, D), lambda i: (i, 0)),
        pl.BlockSpec((1, 1, BLK), lambda i: (i, 0, 0)),
        pl.BlockSpec((1, D), lambda i: (0, 0)),
        pl.BlockSpec((D, D), lambda i: (0, 0)),
    ],
    out_specs=pl.BlockSpec((BLK, D), lambda i: (i, 0)),
    out_shape=jax.ShapeDtypeStruct((N, D), jnp.float32),
)


def _post_body(s_ref, mp_ref, dinv_ref, b_ref, batch_ref, fcw_ref, fcb_ref,
               out_ref, sums, cntb):
    i = pl.program_id(0)

    @pl.when(i == 0)
    def _():
        sums[...] = jnp.zeros_like(sums)
        cntb[...] = jnp.zeros_like(cntb)

    stot = s_ref[0] + s_ref[1] + mp_ref[...]
    dinv = dinv_ref[0, 0, :]
    h = jnp.maximum(dinv[:, None] * stot + b_ref[...], 0.0)
    bb = batch_ref[0, 0, :]
    oh = (bb[:, None] == lax.broadcasted_iota(jnp.int32, (BLK, NUM_GRAPHS), 1))
    ohf = oh.astype(jnp.float32)
    sums[...] += lax.dot_general(ohf, h, (((0,), (0,)), ((), ())),
                                 preferred_element_type=jnp.float32)
    cntb[...] += lax.dot_general(ohf, jnp.ones((BLK, D), jnp.float32),
                                 (((0,), (0,)), ((), ())),
                                 preferred_element_type=jnp.float32)

    @pl.when(i == NBLK - 1)
    def _():
        pooled = sums[...] / jnp.maximum(cntb[...], 1.0)
        out_ref[...] = jnp.dot(pooled, fcw_ref[...],
                               preferred_element_type=jnp.float32) + fcb_ref[...]


_tc_post = pl.pallas_call(
    _post_body,
    grid=(NBLK,),
    in_specs=[
        pl.BlockSpec((NC, BLK, D), lambda i: (0, i, 0)),
        pl.BlockSpec((BLK, D), lambda i: (i, 0)),
        pl.BlockSpec((1, 1, BLK), lambda i: (i, 0, 0)),
        pl.BlockSpec((1, D), lambda i: (0, 0)),
        pl.BlockSpec((1, 1, BLK), lambda i: (i, 0, 0)),
        pl.BlockSpec((D, NUM_CLASSES), lambda i: (0, 0)),
        pl.BlockSpec((1, NUM_CLASSES), lambda i: (0, 0)),
    ],
    out_specs=pl.BlockSpec((NUM_GRAPHS, NUM_CLASSES), lambda i: (0, 0)),
    out_shape=jax.ShapeDtypeStruct((NUM_GRAPHS, NUM_CLASSES), jnp.float32),
    scratch_shapes=[
        pltpu.VMEM((NUM_GRAPHS, D), jnp.float32),
        pltpu.VMEM((NUM_GRAPHS, D), jnp.float32),
    ],
)


def kernel(x, edge_index, batch, emb, W1, b1, W2, b2, W3, b3, fcW, fcb):
    pad = EP - E
    srcf = edge_index[0].astype(jnp.int32)
    dstf = edge_index[1].astype(jnp.int32)
    src_p = jnp.concatenate(
        [srcf, jnp.zeros((pad,), jnp.int32)]).reshape(NWR, IC)
    dst_p = jnp.concatenate(
        [dstf, jnp.full((pad,), N, jnp.int32)]).reshape(NWR, IC)
    x3 = x.astype(jnp.int32).reshape(NBLK, 1, BLK)
    batch3 = batch.astype(jnp.int32).reshape(NBLK, 1, BLK)

    deg_parts = _sc_ones(dst_p)
    dinv3 = _tc_dinv(deg_parts)

    mp1 = _tc_pre(x3, dinv3, emb, W1)
    s1 = _sc_scatter(mp1, src_p, dst_p)
    mp2 = _tc_mid(s1, mp1, dinv3, b1.reshape(1, D), W2)
    s2 = _sc_scatter(mp2, src_p, dst_p)
    mp3 = _tc_mid(s2, mp2, dinv3, b2.reshape(1, D), W3)
    s3 = _sc_scatter(mp3, src_p, dst_p)
    return _tc_post(s3, mp3, dinv3, b3.reshape(1, D), batch3,
                    fcW, fcb.reshape(1, NUM_CLASSES))


# speedup vs baseline: 1.1350x; 1.1350x over previous
import jax
import jax.numpy as jnp
from jax import lax
from jax.experimental import pallas as pl
from jax.experimental.pallas import tpu as pltpu
from jax.experimental.pallas import tpu_sc as plsc

N = 10000
D = 128
E = 320000
NUM_FEAT = 21
NUM_CLASSES = 6
NUM_GRAPHS = 64

NC = 2
NS = 16
NW = NC * NS
IC = 128
RI = 80
EP = NW * RI * IC
N_PAD = 10240
ROWS_PER_TILE = N_PAD // NS
NCH = 5
RCH = RI // NCH

NWR = NW * RI
R0 = 128
R1 = 32

BLK = 1000
NBLK = N // BLK

_mesh = plsc.VectorSubcoreMesh(core_axis_name="c", subcore_axis_name="s")


HF = IC // 2


def _sc_scatter_body(mp_hbm, src_hbm, dst_hbm, out_hbm,
                     sidx, didx, didx64, ga, gb, acc_sh, sema, semb):
    c = lax.axis_index("c")
    s = lax.axis_index("s")
    w = c * NS + s
    row0 = s * ROWS_PER_TILE

    z16 = jnp.zeros((16,), jnp.float32)

    def zero_body(i, carry):
        ga[i // 8, pl.ds((i % 8) * 16, 16)] = z16
        return carry

    lax.fori_loop(0, HF * (D // 16), zero_body, 0)
    for t in range(ROWS_PER_TILE // HF):
        pltpu.sync_copy(ga, acc_sh.at[pl.ds(row0 + t * HF, HF)])
    plsc.subcore_barrier()

    def edge_phase(base, nrows):
      for ch in range(nrows // RCH):
        pltpu.sync_copy(src_hbm.at[pl.ds(base + ch * RCH, RCH)], sidx)
        pltpu.sync_copy(dst_hbm.at[pl.ds(base + ch * RCH, RCH)], didx)

        def pack_body(j, carry):
            didx64[j // 4, pl.ds((j % 4) * 16, 16)] = (
                didx[j // 8, pl.ds((j % 8) * 16, 16)])
            return carry

        lax.fori_loop(0, RCH * (IC // 16), pack_body, 0)

        def src_at(i, h):
            return mp_hbm.at[sidx.at[i, pl.ds(h * HF, HF)]]

        pltpu.async_copy(src_at(0, 0), ga, sema)
        pltpu.async_copy(src_at(0, 1), gb, semb)

        def edge_body(i, carry):
            pltpu.make_async_copy(src_at(i, 0), ga, sema).wait()
            pltpu.sync_copy(ga, acc_sh.at[didx64.at[2 * i]], add=True)

            @pl.when(i < RCH - 1)
            def _():
                pltpu.async_copy(src_at(i + 1, 0), ga, sema)

            pltpu.make_async_copy(src_at(i, 1), gb, semb).wait()
            pltpu.sync_copy(gb, acc_sh.at[didx64.at[2 * i + 1]], add=True)

            @pl.when(i < RCH - 1)
            def _():
                pltpu.async_copy(src_at(i + 1, 1), gb, semb)

            return carry

        lax.fori_loop(0, RCH, edge_body, 0)

    @pl.when(c == 0)
    def _():
        edge_phase(s * R0, R0)

    @pl.when(c == 1)
    def _():
        edge_phase(NS * R0 + s * R1, R1)

    plsc.subcore_barrier()

    for t in range(ROWS_PER_TILE // HF):
        pltpu.sync_copy(acc_sh.at[pl.ds(row0 + t * HF, HF)], ga)
        pltpu.sync_copy(ga, out_hbm.at[c, pl.ds(row0 + t * HF, HF)])


_sc_scatter = pl.kernel(
    _sc_scatter_body,
    out_type=jax.ShapeDtypeStruct((NC, N_PAD, D), jnp.float32),
    mesh=_mesh,
    scratch_types=[
        pltpu.VMEM((RCH, IC), jnp.int32),
        pltpu.VMEM((RCH, IC), jnp.int32),
        pltpu.VMEM((2 * RCH, HF), jnp.int32),
        pltpu.VMEM((HF, D), jnp.float32),
        pltpu.VMEM((HF, D), jnp.float32),
        pltpu.VMEM_SHARED((N_PAD, D), jnp.float32),
        pltpu.SemaphoreType.DMA,
        pltpu.SemaphoreType.DMA,
    ],
)


def _sc_ones_body(dst_hbm, out_hbm, didx, gbuf, acc_sh):
    c = lax.axis_index("c")
    s = lax.axis_index("s")
    w = c * NS + s
    row0 = s * ROWS_PER_TILE

    z16 = jnp.zeros((16,), jnp.float32)

    def zero_body(i, carry):
        gbuf[i // 8, pl.ds((i % 8) * 16, 16)] = z16
        return carry

    lax.fori_loop(0, IC * (D // 16), zero_body, 0)
    for t in range(ROWS_PER_TILE // IC):
        pltpu.sync_copy(gbuf, acc_sh.at[pl.ds(row0 + t * IC, IC)])
    plsc.subcore_barrier()

    one16 = jnp.ones((16,), jnp.float32)

    def ones_body(i, carry):
        gbuf[i // 8, pl.ds((i % 8) * 16, 16)] = one16
        return carry

    lax.fori_loop(0, IC * (D // 16), ones_body, 0)
    for ch in range(NCH):
        pltpu.sync_copy(dst_hbm.at[pl.ds(w * RI + ch * RCH, RCH)], didx)

        def edge_body(r, carry):
            pltpu.sync_copy(gbuf, acc_sh.at[didx.at[r]], add=True)
            return carry

        lax.fori_loop(0, RCH, edge_body, 0)
    plsc.subcore_barrier()

    for t in range(ROWS_PER_TILE // IC):
        pltpu.sync_copy(acc_sh.at[pl.ds(row0 + t * IC, IC)], gbuf)
        pltpu.sync_copy(gbuf, out_hbm.at[c, pl.ds(row0 + t * IC, IC)])


_sc_ones = pl.kernel(
    _sc_ones_body,
    out_type=jax.ShapeDtypeStruct((NC, N_PAD, D), jnp.float32),
    mesh=_mesh,
    scratch_types=[
        pltpu.VMEM((RCH, IC), jnp.int32),
        pltpu.VMEM((IC, D), jnp.float32),
        pltpu.VMEM_SHARED((N_PAD, D), jnp.float32),
    ],
)


def _dinv_body(deg_ref, out_ref):
    cnt = jnp.sum(deg_ref[0] + deg_ref[1], axis=1) * (1.0 / D)
    out_ref[0, 0, :] = lax.rsqrt(cnt + 1.0)


_tc_dinv = pl.pallas_call(
    _dinv_body,
    grid=(NBLK,),
    in_specs=[pl.BlockSpec((NC, BLK, D), lambda i: (0, i, 0))],
    out_specs=pl.BlockSpec((1, 1, BLK), lambda i: (i, 0, 0)),
    out_shape=jax.ShapeDtypeStruct((NBLK, 1, BLK), jnp.float32),
)


def _pre_body(x_ref, dinv_ref, emb_ref, w1_ref, mp_ref):
    xb = x_ref[0, 0, :]
    oh = (xb[:, None] == lax.broadcasted_iota(jnp.int32, (BLK, NUM_FEAT), 1))
    h0 = jnp.dot(oh.astype(jnp.float32), emb_ref[...],
                 preferred_element_type=jnp.float32)
    dinv = dinv_ref[0, 0, :]
    mp_ref[...] = jnp.dot(h0, w1_ref[...],
                          preferred_element_type=jnp.float32) * dinv[:, None]


_tc_pre = pl.pallas_call(
    _pre_body,
    grid=(NBLK,),
    in_specs=[
        pl.BlockSpec((1, 1, BLK), lambda i: (i, 0, 0)),
        pl.BlockSpec((1, 1, BLK), lambda i: (i, 0, 0)),
        pl.BlockSpec((NUM_FEAT, D), lambda i: (0, 0)),
        pl.BlockSpec((D, D), lambda i: (0, 0)),
    ],
    out_specs=pl.BlockSpec((BLK, D), lambda i: (i, 0)),
    out_shape=jax.ShapeDtypeStruct((N, D), jnp.float32),
)


def _mid_body(s_ref, mp_ref, dinv_ref, b_ref, w_ref, out_ref):
    stot = s_ref[0] + s_ref[1] + mp_ref[...]
    dinv = dinv_ref[0, 0, :]
    h = jnp.maximum(dinv[:, None] * stot + b_ref[...], 0.0)
    out_ref[...] = jnp.dot(h, w_ref[...],
                           preferred_element_type=jnp.float32) * dinv[:, None]


_tc_mid = pl.pallas_call(
    _mid_body,
    grid=(NBLK,),
    in_specs=[
        pl.BlockSpec((NC, BLK, D), lambda i: (0, i, 0)),
        pl.BlockSpec((BLK, D), lambda i: (i, 0)),
        pl.BlockSpec((1, 1, BLK), lambda i: (i, 0, 0)),
        pl.BlockSpec((1, D), lambda i: (0, 0)),
        pl.BlockSpec((D, D), lambda i: (0, 0)),
    ],
    out_specs=pl.BlockSpec((BLK, D), lambda i: (i, 0)),
    out_shape=jax.ShapeDtypeStruct((N, D), jnp.float32),
)


def _post_body(s_ref, mp_ref, dinv_ref, b_ref, batch_ref, fcw_ref, fcb_ref,
               out_ref, sums, cntb):
    i = pl.program_id(0)

    @pl.when(i == 0)
    def _():
        sums[...] = jnp.zeros_like(sums)
        cntb[...] = jnp.zeros_like(cntb)

    stot = s_ref[0] + s_ref[1] + mp_ref[...]
    dinv = dinv_ref[0, 0, :]
    h = jnp.maximum(dinv[:, None] * stot + b_ref[...], 0.0)
    bb = batch_ref[0, 0, :]
    oh = (bb[:, None] == lax.broadcasted_iota(jnp.int32, (BLK, NUM_GRAPHS), 1))
    ohf = oh.astype(jnp.float32)
    sums[...] += lax.dot_general(ohf, h, (((0,), (0,)), ((), ())),
                                 preferred_element_type=jnp.float32)
    cntb[...] += lax.dot_general(ohf, jnp.ones((BLK, D), jnp.float32),
                                 (((0,), (0,)), ((), ())),
                                 preferred_element_type=jnp.float32)

    @pl.when(i == NBLK - 1)
    def _():
        pooled = sums[...] / jnp.maximum(cntb[...], 1.0)
        out_ref[...] = jnp.dot(pooled, fcw_ref[...],
                               preferred_element_type=jnp.float32) + fcb_ref[...]


_tc_post = pl.pallas_call(
    _post_body,
    grid=(NBLK,),
    in_specs=[
        pl.BlockSpec((NC, BLK, D), lambda i: (0, i, 0)),
        pl.BlockSpec((BLK, D), lambda i: (i, 0)),
        pl.BlockSpec((1, 1, BLK), lambda i: (i, 0, 0)),
        pl.BlockSpec((1, D), lambda i: (0, 0)),
        pl.BlockSpec((1, 1, BLK), lambda i: (i, 0, 0)),
        pl.BlockSpec((D, NUM_CLASSES), lambda i: (0, 0)),
        pl.BlockSpec((1, NUM_CLASSES), lambda i: (0, 0)),
    ],
    out_specs=pl.BlockSpec((NUM_GRAPHS, NUM_CLASSES), lambda i: (0, 0)),
    out_shape=jax.ShapeDtypeStruct((NUM_GRAPHS, NUM_CLASSES), jnp.float32),
    scratch_shapes=[
        pltpu.VMEM((NUM_GRAPHS, D), jnp.float32),
        pltpu.VMEM((NUM_GRAPHS, D), jnp.float32),
    ],
)


def kernel(x, edge_index, batch, emb, W1, b1, W2, b2, W3, b3, fcW, fcb):
    pad = EP - E
    srcf = edge_index[0].astype(jnp.int32)
    dstf = edge_index[1].astype(jnp.int32)
    src_p = jnp.concatenate(
        [srcf, jnp.zeros((pad,), jnp.int32)]).reshape(NWR, IC)
    dst_p = jnp.concatenate(
        [dstf, jnp.full((pad,), N, jnp.int32)]).reshape(NWR, IC)
    x3 = x.astype(jnp.int32).reshape(NBLK, 1, BLK)
    batch3 = batch.astype(jnp.int32).reshape(NBLK, 1, BLK)

    deg_parts = _sc_ones(dst_p)
    dinv3 = _tc_dinv(deg_parts)

    mp1 = _tc_pre(x3, dinv3, emb, W1)
    s1 = _sc_scatter(mp1, src_p, dst_p)
    mp2 = _tc_mid(s1, mp1, dinv3, b1.reshape(1, D), W2)
    s2 = _sc_scatter(mp2, src_p, dst_p)
    mp3 = _tc_mid(s2, mp2, dinv3, b2.reshape(1, D), W3)
    s3 = _sc_scatter(mp3, src_p, dst_p)
    return _tc_post(s3, mp3, dinv3, b3.reshape(1, D), batch3,
                    fcW, fcb.reshape(1, NUM_CLASSES))


# speedup vs baseline: 1.2215x; 1.0762x over previous
import jax
import jax.numpy as jnp
from jax import lax
from jax.experimental import pallas as pl
from jax.experimental.pallas import tpu as pltpu
from jax.experimental.pallas import tpu_sc as plsc

N = 10000
D = 128
E = 320000
NUM_FEAT = 21
NUM_CLASSES = 6
NUM_GRAPHS = 64

NC = 2
NS = 16
NW = NC * NS
IC = 128
RI = 80
EP = NW * RI * IC
N_PAD = 10240
ROWS_PER_TILE = N_PAD // NS
NCH = 5
RCH = RI // NCH

NWR = NW * RI
R0 = 144
R1 = 16

BLK = 1000
NBLK = N // BLK

_mesh = plsc.VectorSubcoreMesh(core_axis_name="c", subcore_axis_name="s")


HF = IC // 2


def _sc_scatter_body(mp_hbm, src_hbm, dst_hbm, out_hbm,
                     sidx, didx, didx64, ga, gb, acc_sh, sema, semb):
    c = lax.axis_index("c")
    s = lax.axis_index("s")
    w = c * NS + s
    row0 = s * ROWS_PER_TILE

    z16 = jnp.zeros((16,), jnp.float32)

    def zero_body(i, carry):
        ga[i // 8, pl.ds((i % 8) * 16, 16)] = z16
        return carry

    lax.fori_loop(0, HF * (D // 16), zero_body, 0)
    for t in range(ROWS_PER_TILE // HF):
        pltpu.sync_copy(ga, acc_sh.at[pl.ds(row0 + t * HF, HF)])
    plsc.subcore_barrier()

    def edge_phase(base, nrows):
      for ch in range(nrows // RCH):
        pltpu.sync_copy(src_hbm.at[pl.ds(base + ch * RCH, RCH)], sidx)
        pltpu.sync_copy(dst_hbm.at[pl.ds(base + ch * RCH, RCH)], didx)

        def pack_body(j, carry):
            didx64[j // 4, pl.ds((j % 4) * 16, 16)] = (
                didx[j // 8, pl.ds((j % 8) * 16, 16)])
            return carry

        lax.fori_loop(0, RCH * (IC // 16), pack_body, 0)

        def src_at(i, h):
            return mp_hbm.at[sidx.at[i, pl.ds(h * HF, HF)]]

        pltpu.async_copy(src_at(0, 0), ga, sema)
        pltpu.async_copy(src_at(0, 1), gb, semb)

        def edge_body(i, carry):
            pltpu.make_async_copy(src_at(i, 0), ga, sema).wait()
            pltpu.sync_copy(ga, acc_sh.at[didx64.at[2 * i]], add=True)

            @pl.when(i < RCH - 1)
            def _():
                pltpu.async_copy(src_at(i + 1, 0), ga, sema)

            pltpu.make_async_copy(src_at(i, 1), gb, semb).wait()
            pltpu.sync_copy(gb, acc_sh.at[didx64.at[2 * i + 1]], add=True)

            @pl.when(i < RCH - 1)
            def _():
                pltpu.async_copy(src_at(i + 1, 1), gb, semb)

            return carry

        lax.fori_loop(0, RCH, edge_body, 0)

    @pl.when(c == 0)
    def _():
        edge_phase(s * R0, R0)

    @pl.when(c == 1)
    def _():
        edge_phase(NS * R0 + s * R1, R1)

    plsc.subcore_barrier()

    for t in range(ROWS_PER_TILE // HF):
        pltpu.sync_copy(acc_sh.at[pl.ds(row0 + t * HF, HF)], ga)
        pltpu.sync_copy(ga, out_hbm.at[c, pl.ds(row0 + t * HF, HF)])


_sc_scatter = pl.kernel(
    _sc_scatter_body,
    out_type=jax.ShapeDtypeStruct((NC, N_PAD, D), jnp.float32),
    mesh=_mesh,
    scratch_types=[
        pltpu.VMEM((RCH, IC), jnp.int32),
        pltpu.VMEM((RCH, IC), jnp.int32),
        pltpu.VMEM((2 * RCH, HF), jnp.int32),
        pltpu.VMEM((HF, D), jnp.float32),
        pltpu.VMEM((HF, D), jnp.float32),
        pltpu.VMEM_SHARED((N_PAD, D), jnp.float32),
        pltpu.SemaphoreType.DMA,
        pltpu.SemaphoreType.DMA,
    ],
)


def _sc_ones_body(dst_hbm, out_hbm, didx, gbuf, acc_sh):
    c = lax.axis_index("c")
    s = lax.axis_index("s")
    w = c * NS + s
    row0 = s * ROWS_PER_TILE

    z16 = jnp.zeros((16,), jnp.float32)

    def zero_body(i, carry):
        gbuf[i // 8, pl.ds((i % 8) * 16, 16)] = z16
        return carry

    lax.fori_loop(0, IC * (D // 16), zero_body, 0)
    for t in range(ROWS_PER_TILE // IC):
        pltpu.sync_copy(gbuf, acc_sh.at[pl.ds(row0 + t * IC, IC)])
    plsc.subcore_barrier()

    one16 = jnp.ones((16,), jnp.float32)

    def ones_body(i, carry):
        gbuf[i // 8, pl.ds((i % 8) * 16, 16)] = one16
        return carry

    lax.fori_loop(0, IC * (D // 16), ones_body, 0)
    for ch in range(NCH):
        pltpu.sync_copy(dst_hbm.at[pl.ds(w * RI + ch * RCH, RCH)], didx)

        def edge_body(r, carry):
            pltpu.sync_copy(gbuf, acc_sh.at[didx.at[r]], add=True)
            return carry

        lax.fori_loop(0, RCH, edge_body, 0)
    plsc.subcore_barrier()

    for t in range(ROWS_PER_TILE // IC):
        pltpu.sync_copy(acc_sh.at[pl.ds(row0 + t * IC, IC)], gbuf)
        pltpu.sync_copy(gbuf, out_hbm.at[c, pl.ds(row0 + t * IC, IC)])


_sc_ones = pl.kernel(
    _sc_ones_body,
    out_type=jax.ShapeDtypeStruct((NC, N_PAD, D), jnp.float32),
    mesh=_mesh,
    scratch_types=[
        pltpu.VMEM((RCH, IC), jnp.int32),
        pltpu.VMEM((IC, D), jnp.float32),
        pltpu.VMEM_SHARED((N_PAD, D), jnp.float32),
    ],
)


def _dinv_body(deg_ref, out_ref):
    cnt = jnp.sum(deg_ref[0] + deg_ref[1], axis=1) * (1.0 / D)
    out_ref[0, 0, :] = lax.rsqrt(cnt + 1.0)


_tc_dinv = pl.pallas_call(
    _dinv_body,
    grid=(NBLK,),
    in_specs=[pl.BlockSpec((NC, BLK, D), lambda i: (0, i, 0))],
    out_specs=pl.BlockSpec((1, 1, BLK), lambda i: (i, 0, 0)),
    out_shape=jax.ShapeDtypeStruct((NBLK, 1, BLK), jnp.float32),
)


def _pre_body(x_ref, dinv_ref, emb_ref, w1_ref, mp_ref):
    xb = x_ref[0, 0, :]
    oh = (xb[:, None] == lax.broadcasted_iota(jnp.int32, (BLK, NUM_FEAT), 1))
    h0 = jnp.dot(oh.astype(jnp.float32), emb_ref[...],
                 preferred_element_type=jnp.float32)
    dinv = dinv_ref[0, 0, :]
    mp_ref[...] = jnp.dot(h0, w1_ref[...],
                          preferred_element_type=jnp.float32) * dinv[:, None]


_tc_pre = pl.pallas_call(
    _pre_body,
    grid=(NBLK,),
    in_specs=[
        pl.BlockSpec((1, 1, BLK), lambda i: (i, 0, 0)),
        pl.BlockSpec((1, 1, BLK), lambda i: (i, 0, 0)),
        pl.BlockSpec((NUM_FEAT, D), lambda i: (0, 0)),
        pl.BlockSpec((D, D), lambda i: (0, 0)),
    ],
    out_specs=pl.BlockSpec((BLK, D), lambda i: (i, 0)),
    out_shape=jax.ShapeDtypeStruct((N, D), jnp.float32),
)


def _mid_body(s_ref, mp_ref, dinv_ref, b_ref, w_ref, out_ref):
    stot = s_ref[0] + s_ref[1] + mp_ref[...]
    dinv = dinv_ref[0, 0, :]
    h = jnp.maximum(dinv[:, None] * stot + b_ref[...], 0.0)
    out_ref[...] = jnp.dot(h, w_ref[...],
                           preferred_element_type=jnp.float32) * dinv[:, None]


_tc_mid = pl.pallas_call(
    _mid_body,
    grid=(NBLK,),
    in_specs=[
        pl.BlockSpec((NC, BLK, D), lambda i: (0, i, 0)),
        pl.BlockSpec((BLK, D), lambda i: (i, 0)),
        pl.BlockSpec((1, 1, BLK), lambda i: (i, 0, 0)),
        pl.BlockSpec((1, D), lambda i: (0, 0)),
        pl.BlockSpec((D, D), lambda i: (0, 0)),
    ],
    out_specs=pl.BlockSpec((BLK, D), lambda i: (i, 0)),
    out_shape=jax.ShapeDtypeStruct((N, D), jnp.float32),
)


def _post_body(s_ref, mp_ref, dinv_ref, b_ref, batch_ref, fcw_ref, fcb_ref,
               out_ref, sums, cntb):
    i = pl.program_id(0)

    @pl.when(i == 0)
    def _():
        sums[...] = jnp.zeros_like(sums)
        cntb[...] = jnp.zeros_like(cntb)

    stot = s_ref[0] + s_ref[1] + mp_ref[...]
    dinv = dinv_ref[0, 0, :]
    h = jnp.maximum(dinv[:, None] * stot + b_ref[...], 0.0)
    bb = batch_ref[0, 0, :]
    oh = (bb[:, None] == lax.broadcasted_iota(jnp.int32, (BLK, NUM_GRAPHS), 1))
    ohf = oh.astype(jnp.float32)
    sums[...] += lax.dot_general(ohf, h, (((0,), (0,)), ((), ())),
                                 preferred_element_type=jnp.float32)
    cntb[...] += lax.dot_general(ohf, jnp.ones((BLK, D), jnp.float32),
                                 (((0,), (0,)), ((), ())),
                                 preferred_element_type=jnp.float32)

    @pl.when(i == NBLK - 1)
    def _():
        pooled = sums[...] / jnp.maximum(cntb[...], 1.0)
        out_ref[...] = jnp.dot(pooled, fcw_ref[...],
                               preferred_element_type=jnp.float32) + fcb_ref[...]


_tc_post = pl.pallas_call(
    _post_body,
    grid=(NBLK,),
    in_specs=[
        pl.BlockSpec((NC, BLK, D), lambda i: (0, i, 0)),
        pl.BlockSpec((BLK, D), lambda i: (i, 0)),
        pl.BlockSpec((1, 1, BLK), lambda i: (i, 0, 0)),
        pl.BlockSpec((1, D), lambda i: (0, 0)),
        pl.BlockSpec((1, 1, BLK), lambda i: (i, 0, 0)),
        pl.BlockSpec((D, NUM_CLASSES), lambda i: (0, 0)),
        pl.BlockSpec((1, NUM_CLASSES), lambda i: (0, 0)),
    ],
    out_specs=pl.BlockSpec((NUM_GRAPHS, NUM_CLASSES), lambda i: (0, 0)),
    out_shape=jax.ShapeDtypeStruct((NUM_GRAPHS, NUM_CLASSES), jnp.float32),
    scratch_shapes=[
        pltpu.VMEM((NUM_GRAPHS, D), jnp.float32),
        pltpu.VMEM((NUM_GRAPHS, D), jnp.float32),
    ],
)


def kernel(x, edge_index, batch, emb, W1, b1, W2, b2, W3, b3, fcW, fcb):
    pad = EP - E
    srcf = edge_index[0].astype(jnp.int32)
    dstf = edge_index[1].astype(jnp.int32)
    src_p = jnp.concatenate(
        [srcf, jnp.zeros((pad,), jnp.int32)]).reshape(NWR, IC)
    dst_p = jnp.concatenate(
        [dstf, jnp.full((pad,), N, jnp.int32)]).reshape(NWR, IC)
    x3 = x.astype(jnp.int32).reshape(NBLK, 1, BLK)
    batch3 = batch.astype(jnp.int32).reshape(NBLK, 1, BLK)

    deg_parts = _sc_ones(dst_p)
    dinv3 = _tc_dinv(deg_parts)

    mp1 = _tc_pre(x3, dinv3, emb, W1)
    s1 = _sc_scatter(mp1, src_p, dst_p)
    mp2 = _tc_mid(s1, mp1, dinv3, b1.reshape(1, D), W2)
    s2 = _sc_scatter(mp2, src_p, dst_p)
    mp3 = _tc_mid(s2, mp2, dinv3, b2.reshape(1, D), W3)
    s3 = _sc_scatter(mp3, src_p, dst_p)
    return _tc_post(s3, mp3, dinv3, b3.reshape(1, D), batch3,
                    fcW, fcb.reshape(1, NUM_CLASSES))
